# Initial kernel scaffold; baseline (speedup 1.0000x reference)
import jax
import jax.numpy as jnp
from jax.experimental import pallas as pl


def kernel(x, emb_table):
    raise NotImplementedError("write your pallas kernel here")


# speedup vs baseline: 2.5393x; 2.5393x over previous
import math

import jax
import jax.numpy as jnp
from jax import lax
from jax.experimental import pallas as pl

_D = 768
_ROWS_PER_BLOCK = 512


def _pe_add_body(emb_ref, o_ref):
    i = pl.program_id(0)
    r = _ROWS_PER_BLOCK
    pos = (i * r + lax.broadcasted_iota(jnp.int32, (r, _D), 0)).astype(jnp.float32)
    d = lax.broadcasted_iota(jnp.int32, (r, _D), 1)
    half = (d // 2).astype(jnp.float32)
    parity = (d % 2).astype(jnp.float32)
    inv_freq = jnp.exp(half * (-2.0 * math.log(10000.0) / _D))
    ang = pos * inv_freq + parity * (math.pi / 2.0)
    o_ref[...] = emb_ref[...] + jnp.sin(ang)


def kernel(x, emb_table):
    seq_len = x.shape[1]
    out = pl.pallas_call(
        _pe_add_body,
        grid=(seq_len // _ROWS_PER_BLOCK,),
        in_specs=[pl.BlockSpec((_ROWS_PER_BLOCK, _D), lambda i: (i, 0))],
        out_specs=pl.BlockSpec((_ROWS_PER_BLOCK, _D), lambda i: (i, 0)),
        out_shape=jax.ShapeDtypeStruct((seq_len, _D), jnp.float32),
    )(emb_table[:seq_len])
    return out[None]


# speedup vs baseline: 7.4633x; 2.9391x over previous
import math

import jax
import jax.numpy as jnp
from jax import lax
from jax.experimental import pallas as pl

_D = 768
_NB = 32
_ROWS_PER_BLOCK = 512
_A_PER_BLOCK = _ROWS_PER_BLOCK // _NB


def _tables_body(pq_ref, bb_ref):
    na = pq_ref.shape[1]
    d = lax.broadcasted_iota(jnp.int32, (na, _D), 1)
    inv_freq = jnp.exp((d // 2).astype(jnp.float32) * (-2.0 * math.log(10000.0) / _D))
    a = lax.broadcasted_iota(jnp.int32, (na, _D), 0).astype(jnp.float32)
    big_ang = (a * float(_NB)) * inv_freq
    pq_ref[0] = jnp.sin(big_ang)
    pq_ref[1] = jnp.sin(big_ang + math.pi / 2.0)

    nb = bb_ref.shape[1]
    db = lax.broadcasted_iota(jnp.int32, (nb, _D), 1)
    inv_freq_b = jnp.exp((db // 2).astype(jnp.float32) * (-2.0 * math.log(10000.0) / _D))
    parity = (db % 2).astype(jnp.float32)
    b = lax.broadcasted_iota(jnp.int32, (nb, _D), 0).astype(jnp.float32)
    small_ang = b * inv_freq_b + parity * (math.pi / 2.0)
    bb_ref[0] = jnp.sin(small_ang)
    bb_ref[1] = jnp.sin(small_ang + math.pi / 2.0)


def _make_tables(seq_len):
    na = seq_len // _NB
    return pl.pallas_call(
        _tables_body,
        out_shape=(
            jax.ShapeDtypeStruct((2, na, _D), jnp.float32),
            jax.ShapeDtypeStruct((2, _NB, _D), jnp.float32),
        ),
    )()


def _add_body(emb_ref, pq_ref, bb_ref, o_ref):
    i = pl.program_id(0)
    a0 = i * _A_PER_BLOCK
    p = pq_ref[0, pl.ds(a0, _A_PER_BLOCK), :][:, None, :]
    q = pq_ref[1, pl.ds(a0, _A_PER_BLOCK), :][:, None, :]
    sb = bb_ref[0][None, :, :]
    cb = bb_ref[1][None, :, :]
    emb3 = emb_ref[...].reshape(_A_PER_BLOCK, _NB, _D)
    out3 = emb3 + p * cb + q * sb
    o_ref[...] = out3.reshape(_ROWS_PER_BLOCK, _D)


def kernel(x, emb_table):
    seq_len = x.shape[1]
    pq, bb = _make_tables(seq_len)
    na = seq_len // _NB
    out = pl.pallas_call(
        _add_body,
        grid=(seq_len // _ROWS_PER_BLOCK,),
        in_specs=[
            pl.BlockSpec((_ROWS_PER_BLOCK, _D), lambda i: (i, 0)),
            pl.BlockSpec((2, na, _D), lambda i: (0, 0, 0)),
            pl.BlockSpec((2, _NB, _D), lambda i: (0, 0, 0)),
        ],
        out_specs=pl.BlockSpec((_ROWS_PER_BLOCK, _D), lambda i: (i, 0)),
        out_shape=jax.ShapeDtypeStruct((seq_len, _D), jnp.float32),
    )(emb_table[:seq_len], pq, bb)
    return out[None]
